# baseline (speedup 1.0000x reference)
import jax
import jax.numpy as jnp
from jax import lax
from jax.experimental import pallas as pl
from jax.experimental.pallas import tpu as pltpu
from jax.experimental.pallas import tpu_sc as plsc

NC = 2
NS = 16
NW = NC * NS

BATCH = 16384
N_FIELDS = 26
EMBED_DIM = 64
ROWS_W = BATCH // NW
GROWS = 8
NBUF = 4
NGROUP = ROWS_W // GROWS
NITER = NGROUP // NBUF


def _body(idx_hbm, table_hbm, out_hbm, idx_v, rows_v,
          gsem0, gsem1, gsem2, gsem3, osem0, osem1, osem2, osem3):
    wid = lax.axis_index("s") * NC + lax.axis_index("c")
    base = wid * ROWS_W

    pltpu.sync_copy(idx_hbm.at[pl.ds(base, ROWS_W)], idx_v)

    gsems = (gsem0, gsem1, gsem2, gsem3)
    osems = (osem0, osem1, osem2, osem3)

    def gather_desc(g, h, j):
        return pltpu.make_async_copy(
            table_hbm.at[idx_v.at[g * GROWS + j]],
            rows_v.at[h, j],
            gsems[h],
        )

    def out_desc(g, h):
        return pltpu.make_async_copy(
            rows_v.at[h],
            out_hbm.at[pl.ds(base + g * GROWS, GROWS)],
            osems[h],
        )

    def start_gathers(g, h):
        for j in range(GROWS):
            gather_desc(g, h, j).start()

    def wait_gathers(g, h):
        for j in range(GROWS):
            gather_desc(g, h, j).wait()

    start_gathers(0, 0)
    start_gathers(1, 1)

    def loop_body(i, carry):
        g0 = NBUF * i

        wait_gathers(g0, 0)
        out_desc(g0, 0).start()

        @pl.when(i >= 1)
        def _():
            out_desc(g0 - 2, 2).wait()
        start_gathers(g0 + 2, 2)

        wait_gathers(g0 + 1, 1)
        out_desc(g0 + 1, 1).start()

        @pl.when(i >= 1)
        def _():
            out_desc(g0 - 1, 3).wait()
        start_gathers(g0 + 3, 3)

        wait_gathers(g0 + 2, 2)
        out_desc(g0 + 2, 2).start()

        @pl.when(i < NITER - 1)
        def _():
            out_desc(g0, 0).wait()
            start_gathers(g0 + 4, 0)

        wait_gathers(g0 + 3, 3)
        out_desc(g0 + 3, 3).start()

        @pl.when(i < NITER - 1)
        def _():
            out_desc(g0 + 1, 1).wait()
            start_gathers(g0 + 5, 1)

        return carry

    lax.fori_loop(0, NITER, loop_body, 0)

    out_desc(NGROUP - 4, 0).wait()
    out_desc(NGROUP - 3, 1).wait()
    out_desc(NGROUP - 2, 2).wait()
    out_desc(NGROUP - 1, 3).wait()


@jax.jit
def _run(indices, table):
    fn = pl.kernel(
        _body,
        out_type=jax.ShapeDtypeStruct((BATCH, N_FIELDS, EMBED_DIM),
                                      jnp.float32),
        mesh=plsc.VectorSubcoreMesh(core_axis_name="c", subcore_axis_name="s"),
        compiler_params=pltpu.CompilerParams(use_tc_tiling_on_sc=False),
        scratch_types=[
            pltpu.VMEM((ROWS_W, N_FIELDS), jnp.int32),
            pltpu.VMEM((NBUF, GROWS, N_FIELDS, EMBED_DIM), jnp.float32),
        ] + [pltpu.SemaphoreType.DMA] * 8,
    )
    return fn(indices, table)


def kernel(indices, table):
    return _run(indices, table)


# speedup vs baseline: 1.3329x; 1.3329x over previous
import jax
import jax.numpy as jnp
from jax import lax
from jax.experimental import pallas as pl
from jax.experimental.pallas import tpu as pltpu
from jax.experimental.pallas import tpu_sc as plsc

NC = 2
NS = 16
NW = NC * NS

BATCH = 16384
N_FIELDS = 26
EMBED_DIM = 64
ROWS_W = BATCH // NW
LOOK_W = ROWS_W * N_FIELDS
GROWS = 4
LPG = GROWS * N_FIELDS
NBUF = 4
NGROUP = ROWS_W // GROWS
NITER = NGROUP // NBUF


def _body(idx_hbm, table_hbm, out_hbm, idx_v, rows_v,
          gsem0, gsem1, gsem2, gsem3, osem0, osem1, osem2, osem3):
    wid = lax.axis_index("s") * NC + lax.axis_index("c")
    base = wid * ROWS_W

    pltpu.sync_copy(idx_hbm.at[pl.ds(wid * LOOK_W, LOOK_W)], idx_v)

    gsems = (gsem0, gsem1, gsem2, gsem3)
    osems = (osem0, osem1, osem2, osem3)

    def start_pair(g, h):
        for r16 in range(2 * LPG // 16):
            v = idx_v[pl.ds(g * LPG + r16 * 16, 16)]
            for j in range(16):
                kk = r16 * 16 + j
                hh = h + kk // LPG
                kkg = kk % LPG
                pltpu.make_async_copy(
                    table_hbm.at[v[j]],
                    rows_v.at[hh, kkg // N_FIELDS, kkg % N_FIELDS],
                    gsems[hh],
                ).start()

    def wait_gathers(h):
        pltpu.make_async_copy(
            out_hbm.at[pl.ds(0, GROWS)], rows_v.at[h], gsems[h]).wait()

    def out_desc(g, h):
        return pltpu.make_async_copy(
            rows_v.at[h],
            out_hbm.at[pl.ds(base + g * GROWS, GROWS)],
            osems[h],
        )

    start_pair(0, 0)

    def loop_body(i, carry):
        g0 = NBUF * i

        wait_gathers(0)
        out_desc(g0, 0).start()
        wait_gathers(1)
        out_desc(g0 + 1, 1).start()

        @pl.when(i >= 1)
        def _():
            out_desc(g0 - 2, 2).wait()
            out_desc(g0 - 1, 3).wait()
        start_pair(g0 + 2, 2)

        wait_gathers(2)
        out_desc(g0 + 2, 2).start()
        wait_gathers(3)
        out_desc(g0 + 3, 3).start()

        @pl.when(i < NITER - 1)
        def _():
            out_desc(g0, 0).wait()
            out_desc(g0 + 1, 1).wait()
            start_pair(g0 + 4, 0)

        return carry

    lax.fori_loop(0, NITER, loop_body, 0)

    out_desc(NGROUP - 4, 0).wait()
    out_desc(NGROUP - 3, 1).wait()
    out_desc(NGROUP - 2, 2).wait()
    out_desc(NGROUP - 1, 3).wait()


@jax.jit
def _run(indices, table):
    idx_flat = indices.reshape(BATCH * N_FIELDS)
    fn = pl.kernel(
        _body,
        out_type=jax.ShapeDtypeStruct((BATCH, N_FIELDS, EMBED_DIM),
                                      jnp.float32),
        mesh=plsc.VectorSubcoreMesh(core_axis_name="c", subcore_axis_name="s"),
        scratch_types=[
            pltpu.VMEM((LOOK_W,), jnp.int32),
            pltpu.VMEM((NBUF, GROWS, N_FIELDS, EMBED_DIM), jnp.float32),
        ] + [pltpu.SemaphoreType.DMA] * 8,
    )
    return fn(idx_flat, table)


def kernel(indices, table):
    return _run(indices, table)


# speedup vs baseline: 1.3361x; 1.0024x over previous
import jax
import jax.numpy as jnp
from jax import lax
from jax.experimental import pallas as pl
from jax.experimental.pallas import tpu as pltpu
from jax.experimental.pallas import tpu_sc as plsc

NC = 2
NS = 16
NW = NC * NS

BATCH = 16384
N_FIELDS = 26
EMBED_DIM = 64
ROWS_W = BATCH // NW
LOOK_W = ROWS_W * N_FIELDS
GROWS = 4
LPG = GROWS * N_FIELDS
NBUF = 4
NGROUP = ROWS_W // GROWS
NITER = NGROUP // NBUF


def _body(idx_hbm, table_hbm, out_hbm, idx_v, rows_v,
          gsem0, gsem1, gsem2, gsem3, osem0, osem1, osem2, osem3):
    wid = lax.axis_index("s") * NC + lax.axis_index("c")
    base = wid * ROWS_W

    pltpu.sync_copy(idx_hbm.at[pl.ds(wid * LOOK_W, LOOK_W)], idx_v)

    gsems = (gsem0, gsem1, gsem2, gsem3)
    osems = (osem0, osem1, osem2, osem3)

    def start_pair(g, h):
        for r16 in range(2 * LPG // 16):
            v = idx_v[pl.ds(g * LPG + r16 * 16, 16)]
            for j in range(16):
                kk = r16 * 16 + j
                hh = h + kk // LPG
                kkg = kk % LPG
                pltpu.make_async_copy(
                    table_hbm.at[v[j]],
                    rows_v.at[hh, kkg // N_FIELDS, kkg % N_FIELDS],
                    gsems[hh],
                ).start()

    def wait_gathers(h):
        pltpu.make_async_copy(
            out_hbm.at[pl.ds(0, GROWS)], rows_v.at[h], gsems[h]).wait()

    def out_desc(g, h):
        return pltpu.make_async_copy(
            rows_v.at[h],
            out_hbm.at[pl.ds(base + g * GROWS, GROWS)],
            osems[h],
        )

    start_pair(0, 0)

    def loop_body(i, carry):
        g0 = NBUF * i

        wait_gathers(0)
        out_desc(g0, 0).start()
        wait_gathers(1)
        out_desc(g0 + 1, 1).start()

        @pl.when(i >= 1)
        def _():
            out_desc(g0 - 2, 2).wait()
            out_desc(g0 - 1, 3).wait()
        start_pair(g0 + 2, 2)

        wait_gathers(2)
        out_desc(g0 + 2, 2).start()
        wait_gathers(3)
        out_desc(g0 + 3, 3).start()

        @pl.when(i < NITER - 1)
        def _():
            out_desc(g0, 0).wait()
            out_desc(g0 + 1, 1).wait()
            start_pair(g0 + 4, 0)

        return carry

    lax.fori_loop(0, NITER, loop_body, 0)

    out_desc(NGROUP - 4, 0).wait()
    out_desc(NGROUP - 3, 1).wait()
    out_desc(NGROUP - 2, 2).wait()
    out_desc(NGROUP - 1, 3).wait()


@jax.jit
def _run(indices, table):
    idx_flat = indices.reshape(BATCH * N_FIELDS)
    fn = pl.kernel(
        _body,
        out_type=jax.ShapeDtypeStruct((BATCH, N_FIELDS, EMBED_DIM),
                                      jnp.float32),
        mesh=plsc.VectorSubcoreMesh(core_axis_name="c", subcore_axis_name="s"),
        compiler_params=pltpu.CompilerParams(disable_bounds_checks=True),
        scratch_types=[
            pltpu.VMEM((LOOK_W,), jnp.int32),
            pltpu.VMEM((NBUF, GROWS, N_FIELDS, EMBED_DIM), jnp.float32),
        ] + [pltpu.SemaphoreType.DMA] * 8,
    )
    return fn(idx_flat, table)


def kernel(indices, table):
    return _run(indices, table)
